# Initial kernel scaffold; baseline (speedup 1.0000x reference)
import jax
import jax.numpy as jnp
from jax.experimental import pallas as pl


def kernel(inputs, boundaries):
    raise NotImplementedError("write your pallas kernel here")


# speedup vs baseline: 419.6648x; 419.6648x over previous
import functools

import jax
import jax.numpy as jnp
from jax import lax
from jax.experimental import pallas as pl
from jax.experimental.pallas import tpu as pltpu
from jax.experimental.pallas import tpu_sc as plsc

_NC = 2
_NS = 16
_NW = _NC * _NS
_LANES = 16
_TBL = 256
_CHUNK = 16384


def _make_sc_call(n):
    per_w = n // _NW
    n_chunks = per_w // _CHUNK
    mesh = plsc.VectorSubcoreMesh(core_axis_name="c", subcore_axis_name="s")

    @functools.partial(
        pl.kernel,
        out_type=jax.ShapeDtypeStruct((n,), jnp.int32),
        mesh=mesh,
        scratch_types=[
            pltpu.VMEM((_TBL,), jnp.float32),
            pltpu.VMEM((_CHUNK,), jnp.float32),
            pltpu.VMEM((_CHUNK,), jnp.int32),
        ],
        compiler_params=pltpu.CompilerParams(needs_layout_passes=False),
    )
    def run(in_hbm, tbl_hbm, out_hbm, tbl_v, x_v, o_v):
        wid = lax.axis_index("s") * _NC + lax.axis_index("c")
        base = wid * per_w
        pltpu.sync_copy(tbl_hbm, tbl_v)

        def chunk_body(ci, carry):
            off = base + ci * _CHUNK
            pltpu.sync_copy(in_hbm.at[pl.ds(off, _CHUNK)], x_v)

            def vec_body(vi, c):
                x = x_v[pl.ds(vi * _LANES, _LANES)]
                lo = jnp.zeros((_LANES,), jnp.int32)
                for step in (128, 64, 32, 16, 8, 4, 2, 1):
                    g = plsc.load_gather(tbl_v, [lo + (step - 1)])
                    lo = lo + jnp.where(g <= x, jnp.int32(step), jnp.int32(0))
                o_v[pl.ds(vi * _LANES, _LANES)] = lo
                return c

            lax.fori_loop(0, _CHUNK // _LANES, vec_body, 0, unroll=4)
            pltpu.sync_copy(o_v, out_hbm.at[pl.ds(off, _CHUNK)])
            return carry

        lax.fori_loop(0, n_chunks, chunk_body, 0)

    return run


def kernel(inputs, boundaries):
    n = inputs.shape[0]
    tbl = jnp.concatenate(
        [boundaries, jnp.full((1,), jnp.inf, dtype=jnp.float32)]
    )
    return _make_sc_call(n)(inputs, tbl)


# speedup vs baseline: 957.9976x; 2.2828x over previous
import functools

import jax
import jax.numpy as jnp
from jax import lax
from jax.experimental import pallas as pl
from jax.experimental.pallas import tpu as pltpu
from jax.experimental.pallas import tpu_sc as plsc

_NC = 2
_NS = 16
_NW = _NC * _NS
_LANES = 16
_TBL = 256
_CHUNK = 16384


def _make_sc_call(n):
    per_w = n // _NW
    n_chunks = per_w // _CHUNK
    mesh = plsc.VectorSubcoreMesh(core_axis_name="c", subcore_axis_name="s")

    @functools.partial(
        pl.kernel,
        out_type=jax.ShapeDtypeStruct((n,), jnp.int32),
        mesh=mesh,
        scratch_types=[
            pltpu.VMEM((_TBL,), jnp.float32),
            pltpu.VMEM((_CHUNK,), jnp.float32),
            pltpu.VMEM((_CHUNK,), jnp.int32),
        ],
        compiler_params=pltpu.CompilerParams(needs_layout_passes=False),
    )
    def run(in_hbm, tbl_hbm, out_hbm, tbl_v, x_v, o_v):
        wid = lax.axis_index("s") * _NC + lax.axis_index("c")
        base = wid * per_w
        pltpu.sync_copy(tbl_hbm, tbl_v)

        def chunk_body(ci, carry):
            off = base + ci * _CHUNK
            pltpu.sync_copy(in_hbm.at[pl.ds(off, _CHUNK)], x_v)

            @plsc.parallel_loop(0, _CHUNK // _LANES, unroll=8)
            def vec_body(vi):
                x = x_v[pl.ds(vi * _LANES, _LANES)]
                lo = jnp.zeros((_LANES,), jnp.int32)
                for step in (128, 64, 32, 16, 8, 4, 2, 1):
                    g = plsc.load_gather(tbl_v, [lo + (step - 1)])
                    lo = lo + jnp.where(g <= x, jnp.int32(step), jnp.int32(0))
                o_v[pl.ds(vi * _LANES, _LANES)] = lo
            pltpu.sync_copy(o_v, out_hbm.at[pl.ds(off, _CHUNK)])
            return carry

        lax.fori_loop(0, n_chunks, chunk_body, 0)

    return run


def kernel(inputs, boundaries):
    n = inputs.shape[0]
    tbl = jnp.concatenate(
        [boundaries, jnp.full((1,), jnp.inf, dtype=jnp.float32)]
    )
    return _make_sc_call(n)(inputs, tbl)


# speedup vs baseline: 1559.2744x; 1.6276x over previous
import functools

import jax
import jax.numpy as jnp
from jax import lax
from jax.experimental import pallas as pl
from jax.experimental.pallas import tpu as pltpu
from jax.experimental.pallas import tpu_sc as plsc

_NC = 2
_NS = 16
_NW = _NC * _NS
_LANES = 16
_TBL = 256
_CHUNK = 16384


def _search_vreg(x, tbl_v, cs):
    c31, c63, c95, c127, c159, c191, c223 = cs
    m1 = c127 <= x
    lo = jnp.where(m1, jnp.int32(128), jnp.int32(0))
    t2 = jnp.where(m1, c191, c63)
    m2 = t2 <= x
    lo = lo + jnp.where(m2, jnp.int32(64), jnp.int32(0))
    t3 = jnp.where(m2, jnp.where(m1, c223, c95), jnp.where(m1, c159, c31))
    m3 = t3 <= x
    lo = lo + jnp.where(m3, jnp.int32(32), jnp.int32(0))
    for step in (16, 8, 4, 2, 1):
        g = plsc.load_gather(tbl_v, [lo + (step - 1)])
        lo = lo + jnp.where(g <= x, jnp.int32(step), jnp.int32(0))
    return lo


def _make_sc_call(n):
    per_w = n // _NW
    n_chunks = per_w // _CHUNK
    assert n_chunks % 2 == 0
    mesh = plsc.VectorSubcoreMesh(core_axis_name="c", subcore_axis_name="s")

    @functools.partial(
        pl.kernel,
        out_type=jax.ShapeDtypeStruct((n,), jnp.int32),
        mesh=mesh,
        scratch_types=[
            pltpu.VMEM((_TBL,), jnp.float32),
            pltpu.VMEM((_CHUNK,), jnp.float32),
            pltpu.VMEM((_CHUNK,), jnp.float32),
            pltpu.VMEM((_CHUNK,), jnp.int32),
            pltpu.VMEM((_CHUNK,), jnp.int32),
            pltpu.SemaphoreType.DMA,
            pltpu.SemaphoreType.DMA,
            pltpu.SemaphoreType.DMA,
            pltpu.SemaphoreType.DMA,
        ],
        compiler_params=pltpu.CompilerParams(needs_layout_passes=False),
    )
    def run(in_hbm, tbl_hbm, out_hbm, tbl_v, xa, xb, oa, ob,
            in_sa, in_sb, out_sa, out_sb):
        wid = lax.axis_index("s") * _NC + lax.axis_index("c")
        base = wid * per_w
        pltpu.sync_copy(tbl_hbm, tbl_v)

        cs = tuple(
            plsc.load_gather(tbl_v, [jnp.full((_LANES,), i, jnp.int32)])
            for i in (31, 63, 95, 127, 159, 191, 223)
        )

        def in_copy(ci, buf, sem):
            return pltpu.make_async_copy(
                in_hbm.at[pl.ds(base + ci * _CHUNK, _CHUNK)], buf, sem)

        def out_copy(ci, buf, sem):
            return pltpu.make_async_copy(
                buf, out_hbm.at[pl.ds(base + ci * _CHUNK, _CHUNK)], sem)

        def compute(x_v, o_v):
            @plsc.parallel_loop(0, _CHUNK // _LANES, unroll=8)
            def vec_body(vi):
                x = x_v[pl.ds(vi * _LANES, _LANES)]
                o_v[pl.ds(vi * _LANES, _LANES)] = _search_vreg(x, tbl_v, cs)

        in_copy(0, xa, in_sa).start()
        in_copy(1, xb, in_sb).start()

        def body(ct, carry):
            ca = 2 * ct
            cb = 2 * ct + 1
            in_copy(ca, xa, in_sa).wait()

            @pl.when(ct > 0)
            def _():
                out_copy(ca - 2, oa, out_sa).wait()

            compute(xa, oa)
            out_copy(ca, oa, out_sa).start()

            @pl.when(ct + 1 < n_chunks // 2)
            def _():
                in_copy(ca + 2, xa, in_sa).start()

            in_copy(cb, xb, in_sb).wait()

            @pl.when(ct > 0)
            def _():
                out_copy(cb - 2, ob, out_sb).wait()

            compute(xb, ob)
            out_copy(cb, ob, out_sb).start()

            @pl.when(ct + 1 < n_chunks // 2)
            def _():
                in_copy(cb + 2, xb, in_sb).start()

            return carry

        lax.fori_loop(0, n_chunks // 2, body, 0)
        out_copy(n_chunks - 2, oa, out_sa).wait()
        out_copy(n_chunks - 1, ob, out_sb).wait()

    return run


def kernel(inputs, boundaries):
    n = inputs.shape[0]
    tbl = jnp.concatenate(
        [boundaries, jnp.full((1,), jnp.inf, dtype=jnp.float32)]
    )
    return _make_sc_call(n)(inputs, tbl)
